# baseline (speedup 1.0000x reference)
import jax
import jax.numpy as jnp
from jax.experimental import pallas as pl
from jax.experimental.pallas import tpu as pltpu

N = 2048
NUM_EDGE = 5
NUM_CHANNELS = 2
IN_DIM = 256
HIDDEN = 64
NUM_CLASS = 8
WIDE = 384
BM = 128
NBLK = N // BM


def _fused_kernel(filt_ref, a_ref, t0_ref, gw_ref, gb_ref, l1w_ref, l1b_ref,
                  l2w_ref, l2b_ref, y_ref, q_scr, t_scr):
    p = pl.program_id(0)
    blk = pl.program_id(1)
    rows = pl.ds(blk * BM, BM)

    @pl.when(p == 0)
    def _phase0():
        ab = [a_ref[r].astype(jnp.bfloat16) for r in range(NUM_EDGE)]
        B = [jnp.dot(ab[r], t0_ref[...], preferred_element_type=jnp.float32)
             for r in range(NUM_EDGE)]
        for c in range(NUM_CHANNELS):
            t = filt_ref[2, c, 0] * B[0]
            for r in range(1, NUM_EDGE):
                t = t + filt_ref[2, c, r] * B[r]
            t_scr[0, c, rows, :] = t.astype(jnp.bfloat16)
            for s in (1, 0):
                acc = filt_ref[s, c, 0].astype(jnp.bfloat16) * ab[0]
                for r in range(1, NUM_EDGE):
                    acc = acc + filt_ref[s, c, r].astype(jnp.bfloat16) * ab[r]
                q_scr[s, c, rows, :] = acc

    @pl.when(p == 1)
    def _phase1():
        for c in range(NUM_CHANNELS):
            prod = jnp.dot(q_scr[1, c, rows, :], t_scr[0, c],
                           preferred_element_type=jnp.float32)
            t_scr[1, c, rows, :] = prod.astype(jnp.bfloat16)

    @pl.when(p == 2)
    def _phase2():
        xs = []
        for c in range(NUM_CHANNELS):
            v = jnp.dot(q_scr[0, c, rows, :], t_scr[1, c],
                        preferred_element_type=jnp.float32)
            num = v[:, :IN_DIM]
            e = v[:, IN_DIM:IN_DIM + 1]
            agg = num * (1.0 / jnp.where(e == 0.0, 1.0, e))
            x = jnp.dot(agg, gw_ref[...], preferred_element_type=jnp.float32)
            xs.append(jnp.maximum(x + gb_ref[...], 0.0))
        z = (jnp.dot(xs[0], l1w_ref[:HIDDEN], preferred_element_type=jnp.float32)
             + jnp.dot(xs[1], l1w_ref[HIDDEN:], preferred_element_type=jnp.float32)
             + l1b_ref[...])
        z = jnp.maximum(z, 0.0)
        y_ref[...] = (jnp.dot(z, l2w_ref[...], preferred_element_type=jnp.float32)
                      + l2b_ref[...])


def kernel(A, h, W_conv, gcn_w, gcn_b, lin1_w, lin1_b, lin2_w, lin2_b):
    filt = jax.nn.softmax(W_conv, axis=2)
    t0 = jnp.concatenate(
        [h, jnp.ones((N, 1), jnp.float32),
         jnp.zeros((N, WIDE - IN_DIM - 1), jnp.float32)],
        axis=1).astype(jnp.bfloat16)
    small = lambda shp: pl.BlockSpec(shp, lambda p, i: tuple(0 for _ in shp))
    return pl.pallas_call(
        _fused_kernel,
        grid=(3, NBLK),
        in_specs=[
            pl.BlockSpec(memory_space=pltpu.SMEM),
            pl.BlockSpec((NUM_EDGE, BM, N),
                         lambda p, i: (0, jnp.where(p == 0, i, NBLK - 1), 0)),
            pl.BlockSpec((N, WIDE), lambda p, i: (0, 0)),
            small((IN_DIM, HIDDEN)),
            small((1, HIDDEN)),
            small((NUM_CHANNELS * HIDDEN, HIDDEN)),
            small((1, HIDDEN)),
            small((HIDDEN, NUM_CLASS)),
            small((1, NUM_CLASS)),
        ],
        out_specs=pl.BlockSpec((BM, NUM_CLASS),
                               lambda p, i: (jnp.where(p == 2, i, 0), 0)),
        out_shape=jax.ShapeDtypeStruct((N, NUM_CLASS), jnp.float32),
        scratch_shapes=[
            pltpu.VMEM((2, NUM_CHANNELS, N, N), jnp.bfloat16),
            pltpu.VMEM((2, NUM_CHANNELS, N, WIDE), jnp.bfloat16),
        ],
    )(filt, A, t0, gcn_w, gcn_b.reshape(1, HIDDEN),
      lin1_w, lin1_b.reshape(1, HIDDEN),
      lin2_w, lin2_b.reshape(1, NUM_CLASS))


# speedup vs baseline: 1.2857x; 1.2857x over previous
import jax
import jax.numpy as jnp
from jax.experimental import pallas as pl
from jax.experimental.pallas import tpu as pltpu

N = 2048
NUM_EDGE = 5
NUM_CHANNELS = 2
IN_DIM = 256
HIDDEN = 64
NUM_CLASS = 8
WIDE = 384
BM = 128
NBLK = N // BM
BW = 512
NWBLK = N // BW
STEPS = NBLK + 2 * NWBLK


def _fused_kernel(filt_ref, a_ref, t0_ref, gw_ref, gb_ref, l1w_ref, l1b_ref,
                  l2w_ref, l2b_ref, y_ref, q_scr, t_scr):
    i = pl.program_id(0)

    @pl.when(i < NBLK)
    def _phase0():
        rows = pl.ds(i * BM, BM)
        ab = [a_ref[r].astype(jnp.bfloat16) for r in range(NUM_EDGE)]

        def comb(s, c):
            acc = filt_ref[s, c, 0].astype(jnp.bfloat16) * ab[0]
            for r in range(1, NUM_EDGE):
                acc = acc + filt_ref[s, c, r].astype(jnp.bfloat16) * ab[r]
            return acc

        for c in range(NUM_CHANNELS):
            q_scr[1, c, rows, :] = comb(1, c)
            q_scr[0, c, rows, :] = comb(0, c)
        q2 = jnp.concatenate([comb(2, c) for c in range(NUM_CHANNELS)], axis=0)
        t = jnp.dot(q2, t0_ref[...], preferred_element_type=jnp.float32)
        for c in range(NUM_CHANNELS):
            t_scr[0, c, rows, :] = t[c * BM:(c + 1) * BM].astype(jnp.bfloat16)

    @pl.when(jnp.logical_and(i >= NBLK, i < NBLK + NWBLK))
    def _phase1():
        j = i - NBLK
        rows = pl.ds(j * BW, BW)
        for c in range(NUM_CHANNELS):
            prod = jnp.dot(q_scr[1, c, rows, :], t_scr[0, c],
                           preferred_element_type=jnp.float32)
            t_scr[1, c, rows, :] = prod.astype(jnp.bfloat16)

    @pl.when(i >= NBLK + NWBLK)
    def _phase2():
        j = i - (NBLK + NWBLK)
        rows = pl.ds(j * BW, BW)
        xs = []
        for c in range(NUM_CHANNELS):
            v = jnp.dot(q_scr[0, c, rows, :], t_scr[1, c],
                        preferred_element_type=jnp.float32)
            num = v[:, :IN_DIM]
            e = v[:, IN_DIM:IN_DIM + 1]
            agg = num * (1.0 / jnp.where(e == 0.0, 1.0, e))
            x = jnp.dot(agg, gw_ref[...], preferred_element_type=jnp.float32)
            xs.append(jnp.maximum(x + gb_ref[...], 0.0))
        z = (jnp.dot(xs[0], l1w_ref[:HIDDEN], preferred_element_type=jnp.float32)
             + jnp.dot(xs[1], l1w_ref[HIDDEN:], preferred_element_type=jnp.float32)
             + l1b_ref[...])
        z = jnp.maximum(z, 0.0)
        y_ref[...] = (jnp.dot(z, l2w_ref[...], preferred_element_type=jnp.float32)
                      + l2b_ref[...])


def kernel(A, h, W_conv, gcn_w, gcn_b, lin1_w, lin1_b, lin2_w, lin2_b):
    filt = jax.nn.softmax(W_conv, axis=2)
    t0 = jnp.concatenate(
        [h, jnp.ones((N, 1), jnp.float32),
         jnp.zeros((N, WIDE - IN_DIM - 1), jnp.float32)],
        axis=1).astype(jnp.bfloat16)
    small = lambda shp: pl.BlockSpec(shp, lambda i: tuple(0 for _ in shp))
    return pl.pallas_call(
        _fused_kernel,
        grid=(STEPS,),
        in_specs=[
            pl.BlockSpec(memory_space=pltpu.SMEM),
            pl.BlockSpec((NUM_EDGE, BM, N),
                         lambda i: (0, jnp.where(i < NBLK, i, NBLK - 1), 0)),
            pl.BlockSpec((N, WIDE), lambda i: (0, 0)),
            small((IN_DIM, HIDDEN)),
            small((1, HIDDEN)),
            small((NUM_CHANNELS * HIDDEN, HIDDEN)),
            small((1, HIDDEN)),
            small((HIDDEN, NUM_CLASS)),
            small((1, NUM_CLASS)),
        ],
        out_specs=pl.BlockSpec(
            (BW, NUM_CLASS),
            lambda i: (jnp.where(i >= NBLK + NWBLK, i - (NBLK + NWBLK), 0), 0)),
        out_shape=jax.ShapeDtypeStruct((N, NUM_CLASS), jnp.float32),
        scratch_shapes=[
            pltpu.VMEM((2, NUM_CHANNELS, N, N), jnp.bfloat16),
            pltpu.VMEM((2, NUM_CHANNELS, N, WIDE), jnp.bfloat16),
        ],
    )(filt, A, t0, gcn_w, gcn_b.reshape(1, HIDDEN),
      lin1_w, lin1_b.reshape(1, HIDDEN),
      lin2_w, lin2_b.reshape(1, NUM_CLASS))


# speedup vs baseline: 1.3522x; 1.0517x over previous
import jax
import jax.numpy as jnp
from jax.experimental import pallas as pl
from jax.experimental.pallas import tpu as pltpu

N = 2048
NUM_EDGE = 5
NUM_CHANNELS = 2
IN_DIM = 256
HIDDEN = 64
NUM_CLASS = 8
WIDE = 128
BM = 128
NBLK = N // BM
BW = 512
NWBLK = N // BW
STEPS = 1 + NBLK + 2 * NWBLK


def _fused_kernel(filt_ref, a_ref, h_ref, gwp_ref, gb_ref, l1w_ref, l1b_ref,
                  l2w_ref, l2b_ref, y_ref, q_scr, t_scr, t0_scr):
    i = pl.program_id(0)

    @pl.when(i == 0)
    def _prep():
        hg = jnp.dot(h_ref[...], gwp_ref[...],
                     preferred_element_type=jnp.float32)
        lane = jax.lax.broadcasted_iota(jnp.int32, (N, WIDE), 1)
        t0_scr[...] = (hg + jnp.where(lane == HIDDEN, 1.0, 0.0)
                       ).astype(jnp.bfloat16)

    @pl.when(jnp.logical_and(i >= 1, i < 1 + NBLK))
    def _phase0():
        blk = i - 1
        rows = pl.ds(blk * BM, BM)
        ab = [a_ref[r].astype(jnp.bfloat16) for r in range(NUM_EDGE)]

        def comb(s, c):
            acc = filt_ref[s, c, 0].astype(jnp.bfloat16) * ab[0]
            for r in range(1, NUM_EDGE):
                acc = acc + filt_ref[s, c, r].astype(jnp.bfloat16) * ab[r]
            return acc

        for c in range(NUM_CHANNELS):
            q_scr[1, c, rows, :] = comb(1, c)
            q_scr[0, c, rows, :] = comb(0, c)
        q2 = jnp.concatenate([comb(2, c) for c in range(NUM_CHANNELS)], axis=0)
        t = jnp.dot(q2, t0_scr[...], preferred_element_type=jnp.float32)
        for c in range(NUM_CHANNELS):
            t_scr[0, c, rows, :] = t[c * BM:(c + 1) * BM].astype(jnp.bfloat16)

    @pl.when(jnp.logical_and(i >= 1 + NBLK, i < 1 + NBLK + NWBLK))
    def _phase1():
        j = i - (1 + NBLK)
        rows = pl.ds(j * BW, BW)
        for c in range(NUM_CHANNELS):
            prod = jnp.dot(q_scr[1, c, rows, :], t_scr[0, c],
                           preferred_element_type=jnp.float32)
            t_scr[1, c, rows, :] = prod.astype(jnp.bfloat16)

    @pl.when(i >= 1 + NBLK + NWBLK)
    def _phase2():
        j = i - (1 + NBLK + NWBLK)
        rows = pl.ds(j * BW, BW)
        xs = []
        for c in range(NUM_CHANNELS):
            v = jnp.dot(q_scr[0, c, rows, :], t_scr[1, c],
                        preferred_element_type=jnp.float32)
            numg = v[:, :HIDDEN]
            e = v[:, HIDDEN:HIDDEN + 1]
            x = numg * (1.0 / jnp.where(e == 0.0, 1.0, e))
            xs.append(jnp.maximum(x + gb_ref[...], 0.0))
        z = (jnp.dot(xs[0], l1w_ref[:HIDDEN], preferred_element_type=jnp.float32)
             + jnp.dot(xs[1], l1w_ref[HIDDEN:], preferred_element_type=jnp.float32)
             + l1b_ref[...])
        z = jnp.maximum(z, 0.0)
        y_ref[...] = (jnp.dot(z, l2w_ref[...], preferred_element_type=jnp.float32)
                      + l2b_ref[...])


def kernel(A, h, W_conv, gcn_w, gcn_b, lin1_w, lin1_b, lin2_w, lin2_b):
    filt = jax.nn.softmax(W_conv, axis=2)
    gwp = jnp.pad(gcn_w, ((0, 0), (0, WIDE - HIDDEN))).astype(jnp.bfloat16)
    small = lambda shp: pl.BlockSpec(shp, lambda i: tuple(0 for _ in shp))
    return pl.pallas_call(
        _fused_kernel,
        grid=(STEPS,),
        in_specs=[
            pl.BlockSpec(memory_space=pltpu.SMEM),
            pl.BlockSpec(
                (NUM_EDGE, BM, N),
                lambda i: (0, jnp.clip(i - 1, 0, NBLK - 1), 0)),
            pl.BlockSpec((N, IN_DIM), lambda i: (0, 0)),
            small((IN_DIM, WIDE)),
            small((1, HIDDEN)),
            small((NUM_CHANNELS * HIDDEN, HIDDEN)),
            small((1, HIDDEN)),
            small((HIDDEN, NUM_CLASS)),
            small((1, NUM_CLASS)),
        ],
        out_specs=pl.BlockSpec(
            (BW, NUM_CLASS),
            lambda i: (jnp.where(i >= 1 + NBLK + NWBLK,
                                 i - (1 + NBLK + NWBLK), 0), 0)),
        out_shape=jax.ShapeDtypeStruct((N, NUM_CLASS), jnp.float32),
        scratch_shapes=[
            pltpu.VMEM((2, NUM_CHANNELS, N, N), jnp.bfloat16),
            pltpu.VMEM((2, NUM_CHANNELS, N, WIDE), jnp.bfloat16),
            pltpu.VMEM((N, WIDE), jnp.bfloat16),
        ],
    )(filt, A, h.astype(jnp.bfloat16), gwp, gcn_b.reshape(1, HIDDEN),
      lin1_w, lin1_b.reshape(1, HIDDEN),
      lin2_w, lin2_b.reshape(1, NUM_CLASS))


# speedup vs baseline: 1.4028x; 1.0374x over previous
import jax
import jax.numpy as jnp
from jax.experimental import pallas as pl
from jax.experimental.pallas import tpu as pltpu

N = 2048
NUM_EDGE = 5
NUM_CHANNELS = 2
IN_DIM = 256
HIDDEN = 64
NUM_CLASS = 8
WIDE = 128
BM = 128
NBLK = N // BM
BW = 2048
NWBLK = N // BW
STEPS = 1 + NBLK + 2 * NWBLK


def _fused_kernel(filt_ref, a_ref, h_ref, gwp_ref, gb_ref, l1w_ref, l1b_ref,
                  l2w_ref, l2b_ref, y_ref, q_scr, t_scr, t0_scr):
    i = pl.program_id(0)

    @pl.when(i == 0)
    def _prep():
        hg = jnp.dot(h_ref[...], gwp_ref[...],
                     preferred_element_type=jnp.float32)
        lane = jax.lax.broadcasted_iota(jnp.int32, (N, WIDE), 1)
        t0_scr[...] = (hg + jnp.where(lane == HIDDEN, 1.0, 0.0)
                       ).astype(jnp.bfloat16)

    @pl.when(jnp.logical_and(i >= 1, i < 1 + NBLK))
    def _phase0():
        blk = i - 1
        rows = pl.ds(blk * BM, BM)
        ab = [a_ref[r].astype(jnp.bfloat16) for r in range(NUM_EDGE)]

        def comb(s, c):
            acc = filt_ref[s, c, 0].astype(jnp.bfloat16) * ab[0]
            for r in range(1, NUM_EDGE):
                acc = acc + filt_ref[s, c, r].astype(jnp.bfloat16) * ab[r]
            return acc

        for c in range(NUM_CHANNELS):
            q_scr[1, c, rows, :] = comb(1, c)
            q_scr[0, c, rows, :] = comb(0, c)
        q2 = jnp.concatenate([comb(2, c) for c in range(NUM_CHANNELS)], axis=0)
        t = jnp.dot(q2, t0_scr[...], preferred_element_type=jnp.float32)
        for c in range(NUM_CHANNELS):
            t_scr[0, c, rows, :] = t[c * BM:(c + 1) * BM].astype(jnp.bfloat16)

    @pl.when(jnp.logical_and(i >= 1 + NBLK, i < 1 + NBLK + NWBLK))
    def _phase1():
        j = i - (1 + NBLK)
        rows = pl.ds(j * BW, BW)
        for c in range(NUM_CHANNELS):
            prod = jnp.dot(q_scr[1, c, rows, :], t_scr[0, c],
                           preferred_element_type=jnp.float32)
            t_scr[1, c, rows, :] = prod.astype(jnp.bfloat16)

    @pl.when(i >= 1 + NBLK + NWBLK)
    def _phase2():
        j = i - (1 + NBLK + NWBLK)
        rows = pl.ds(j * BW, BW)
        xs = []
        for c in range(NUM_CHANNELS):
            v = jnp.dot(q_scr[0, c, rows, :], t_scr[1, c],
                        preferred_element_type=jnp.float32)
            numg = v[:, :HIDDEN]
            e = v[:, HIDDEN:HIDDEN + 1]
            x = numg * (1.0 / jnp.where(e == 0.0, 1.0, e))
            xs.append(jnp.maximum(x + gb_ref[...], 0.0))
        z = (jnp.dot(xs[0], l1w_ref[:HIDDEN], preferred_element_type=jnp.float32)
             + jnp.dot(xs[1], l1w_ref[HIDDEN:], preferred_element_type=jnp.float32)
             + l1b_ref[...])
        z = jnp.maximum(z, 0.0)
        y_ref[...] = (jnp.dot(z, l2w_ref[...], preferred_element_type=jnp.float32)
                      + l2b_ref[...])


def kernel(A, h, W_conv, gcn_w, gcn_b, lin1_w, lin1_b, lin2_w, lin2_b):
    filt = jax.nn.softmax(W_conv, axis=2)
    gwp = jnp.pad(gcn_w, ((0, 0), (0, WIDE - HIDDEN))).astype(jnp.bfloat16)
    small = lambda shp: pl.BlockSpec(shp, lambda i: tuple(0 for _ in shp))
    return pl.pallas_call(
        _fused_kernel,
        grid=(STEPS,),
        in_specs=[
            pl.BlockSpec(memory_space=pltpu.SMEM),
            pl.BlockSpec(
                (NUM_EDGE, BM, N),
                lambda i: (0, jnp.clip(i - 1, 0, NBLK - 1), 0)),
            pl.BlockSpec((N, IN_DIM), lambda i: (0, 0)),
            small((IN_DIM, WIDE)),
            small((1, HIDDEN)),
            small((NUM_CHANNELS * HIDDEN, HIDDEN)),
            small((1, HIDDEN)),
            small((HIDDEN, NUM_CLASS)),
            small((1, NUM_CLASS)),
        ],
        out_specs=pl.BlockSpec(
            (BW, NUM_CLASS),
            lambda i: (jnp.where(i >= 1 + NBLK + NWBLK,
                                 i - (1 + NBLK + NWBLK), 0), 0)),
        out_shape=jax.ShapeDtypeStruct((N, NUM_CLASS), jnp.float32),
        scratch_shapes=[
            pltpu.VMEM((2, NUM_CHANNELS, N, N), jnp.bfloat16),
            pltpu.VMEM((2, NUM_CHANNELS, N, WIDE), jnp.bfloat16),
            pltpu.VMEM((N, WIDE), jnp.bfloat16),
        ],
    )(filt, A, h.astype(jnp.bfloat16), gwp, gcn_b.reshape(1, HIDDEN),
      lin1_w, lin1_b.reshape(1, HIDDEN),
      lin2_w, lin2_b.reshape(1, NUM_CLASS))


# speedup vs baseline: 1.4218x; 1.0135x over previous
import jax
import jax.numpy as jnp
from jax.experimental import pallas as pl
from jax.experimental.pallas import tpu as pltpu

N = 2048
NUM_EDGE = 5
NUM_CHANNELS = 2
IN_DIM = 256
HIDDEN = 64
NUM_CLASS = 8
WIDE = 128
BM = 128
NBLK = N // BM
BW = 2048
NWBLK = N // BW
STEPS = NBLK + 2 * NWBLK


def _fused_kernel(filt_ref, a_ref, h_ref, gwp_ref, gb_ref, l1w_ref, l1b_ref,
                  l2w_ref, l2b_ref, y_ref, q_scr, t_scr, t0_scr):
    i = pl.program_id(0)

    @pl.when(i == 0)
    def _prep():
        hg = jnp.dot(h_ref[...], gwp_ref[...],
                     preferred_element_type=jnp.float32)
        lane = jax.lax.broadcasted_iota(jnp.int32, (N, WIDE), 1)
        t0_scr[...] = (hg + jnp.where(lane == HIDDEN, 1.0, 0.0)
                       ).astype(jnp.bfloat16)

    @pl.when(i < NBLK)
    def _phase0():
        rows = pl.ds(i * BM, BM)
        ab = [a_ref[r].astype(jnp.bfloat16) for r in range(NUM_EDGE)]

        def comb(s, c):
            acc = filt_ref[s, c, 0].astype(jnp.bfloat16) * ab[0]
            for r in range(1, NUM_EDGE):
                acc = acc + filt_ref[s, c, r].astype(jnp.bfloat16) * ab[r]
            return acc

        for c in range(NUM_CHANNELS):
            q_scr[1, c, rows, :] = comb(1, c)
            q_scr[0, c, rows, :] = comb(0, c)
        q2 = jnp.concatenate([comb(2, c) for c in range(NUM_CHANNELS)], axis=0)
        t = jnp.dot(q2, t0_scr[...], preferred_element_type=jnp.float32)
        for c in range(NUM_CHANNELS):
            t_scr[0, c, rows, :] = t[c * BM:(c + 1) * BM].astype(jnp.bfloat16)

    @pl.when(jnp.logical_and(i >= NBLK, i < NBLK + NWBLK))
    def _phase1():
        j = i - NBLK
        rows = pl.ds(j * BW, BW)
        for c in range(NUM_CHANNELS):
            prod = jnp.dot(q_scr[1, c, rows, :], t_scr[0, c],
                           preferred_element_type=jnp.float32)
            t_scr[1, c, rows, :] = prod.astype(jnp.bfloat16)

    @pl.when(i >= NBLK + NWBLK)
    def _phase2():
        j = i - (NBLK + NWBLK)
        rows = pl.ds(j * BW, BW)
        xs = []
        for c in range(NUM_CHANNELS):
            v = jnp.dot(q_scr[0, c, rows, :], t_scr[1, c],
                        preferred_element_type=jnp.float32)
            numg = v[:, :HIDDEN]
            e = v[:, HIDDEN:HIDDEN + 1]
            x = numg * (1.0 / jnp.where(e == 0.0, 1.0, e))
            xs.append(jnp.maximum(x + gb_ref[...], 0.0))
        z = (jnp.dot(xs[0], l1w_ref[:HIDDEN], preferred_element_type=jnp.float32)
             + jnp.dot(xs[1], l1w_ref[HIDDEN:], preferred_element_type=jnp.float32)
             + l1b_ref[...])
        z = jnp.maximum(z, 0.0)
        y_ref[...] = (jnp.dot(z, l2w_ref[...], preferred_element_type=jnp.float32)
                      + l2b_ref[...])


def kernel(A, h, W_conv, gcn_w, gcn_b, lin1_w, lin1_b, lin2_w, lin2_b):
    filt = jax.nn.softmax(W_conv, axis=2)
    gwp = jnp.pad(gcn_w, ((0, 0), (0, WIDE - HIDDEN))).astype(jnp.bfloat16)
    small = lambda shp: pl.BlockSpec(shp, lambda i: tuple(0 for _ in shp))
    return pl.pallas_call(
        _fused_kernel,
        grid=(STEPS,),
        in_specs=[
            pl.BlockSpec(memory_space=pltpu.SMEM),
            pl.BlockSpec(
                (NUM_EDGE, BM, N),
                lambda i: (0, jnp.clip(i, 0, NBLK - 1), 0)),
            pl.BlockSpec((N, IN_DIM), lambda i: (0, 0)),
            small((IN_DIM, WIDE)),
            small((1, HIDDEN)),
            small((NUM_CHANNELS * HIDDEN, HIDDEN)),
            small((1, HIDDEN)),
            small((HIDDEN, NUM_CLASS)),
            small((1, NUM_CLASS)),
        ],
        out_specs=pl.BlockSpec(
            (BW, NUM_CLASS),
            lambda i: (jnp.where(i >= NBLK + NWBLK,
                                 i - (NBLK + NWBLK), 0), 0)),
        out_shape=jax.ShapeDtypeStruct((N, NUM_CLASS), jnp.float32),
        scratch_shapes=[
            pltpu.VMEM((2, NUM_CHANNELS, N, N), jnp.bfloat16),
            pltpu.VMEM((2, NUM_CHANNELS, N, WIDE), jnp.bfloat16),
            pltpu.VMEM((N, WIDE), jnp.bfloat16),
        ],
    )(filt, A, h.astype(jnp.bfloat16), gwp, gcn_b.reshape(1, HIDDEN),
      lin1_w, lin1_b.reshape(1, HIDDEN),
      lin2_w, lin2_b.reshape(1, NUM_CLASS))
